# baseline (speedup 1.0000x reference)
import functools

import jax
import jax.numpy as jnp
from jax.experimental import pallas as pl
from jax.experimental.pallas import tpu as pltpu

_PI = 3.1415926235897933
_GAMMA = 12.0
_EMB_RANGE = 0.875
_SCALE = _EMB_RANGE / _PI
_D = 16
_B = 16
_Q = 1
_TN = 16384


def _precompute_body(e1_ref, rel_ref, emb_e_ref, emb_rel_ref,
                     su_ref, cu_ref, wq_ref):
    head = emb_e_ref[0]
    r = emb_rel_ref[0]
    ph = head[:, :_D]
    mh = head[:, _D:]
    pr = r[:, :_D]
    mr = jnp.abs(r[:, _D:2 * _D])
    br = jnp.minimum(r[:, 2 * _D:], 1.0)
    br = jnp.where(br < -mr, -mr, br)
    u = (ph + pr) * (0.5 / _SCALE)
    su_ref[0] = jnp.sin(u)
    cu_ref[0] = jnp.cos(u)
    a = mh * (mr + br)
    c = 1.0 - br
    wq_ref[0] = jnp.concatenate([-2.0 * a * c, c * c, a * a], axis=1)


def _score_quarter(tail, su_b, cu_b, w, sa2, pw, mw):
    tail_t = tail.T
    v = tail_t[:_D] * (0.5 / _SCALE)
    v2 = v * v
    sv = v * (1.0 + v2 * (-1.0 / 6.0 + v2 * (1.0 / 120.0 + v2 * (-1.0 / 5040.0
         + v2 * (1.0 / 362880.0)))))
    cv = 1.0 + v2 * (-0.5 + v2 * (1.0 / 24.0 + v2 * (-1.0 / 720.0
         + v2 * (1.0 / 40320.0))))
    sv_b = sv.astype(jnp.bfloat16)
    cv_b = cv.astype(jnp.bfloat16)
    mt = tail_t[_D:]
    x_mat = jnp.concatenate([mt, mt * mt], axis=0).astype(jnp.bfloat16)

    term = (su_b[:, :, None] * cv_b[None, :, :]
            - cu_b[:, :, None] * sv_b[None, :, :])
    abs_t = jnp.abs(term).reshape(_B * _D, term.shape[-1])

    row = jax.lax.broadcasted_iota(jnp.int32, (_B, _B * _D), 0)
    col = jax.lax.broadcasted_iota(jnp.int32, (_B, _B * _D), 1)
    mask = (col // _D == row).astype(jnp.bfloat16)
    phase = jnp.dot(mask, abs_t, preferred_element_type=jnp.float32)

    quad = sa2 + jnp.dot(w.astype(jnp.bfloat16), x_mat,
                         preferred_element_type=jnp.float32)
    r_score = jnp.sqrt(jnp.maximum(quad, 0.0))

    x = _GAMMA - (phase * pw + r_score * mw)
    return jax.nn.sigmoid(x)


def _make_score_body(num_ents):
    n_blocks = (num_ents + _TN - 1) // _TN
    tail_rows = num_ents - (n_blocks - 1) * _TN

    def _copy(emb_ref, buf, sems, blk, slot, partial):
        if partial:
            return pltpu.make_async_copy(
                emb_ref.at[pl.ds(blk * _TN, tail_rows)],
                buf.at[slot, pl.ds(0, tail_rows)],
                sems.at[slot])
        return pltpu.make_async_copy(
            emb_ref.at[pl.ds(blk * _TN, _TN)],
            buf.at[slot], sems.at[slot])

    def _score_body(su_ref, cu_ref, wq_ref, pw_ref, mw_ref, emb_ref,
                    out_ref, buf, sems):
        i = pl.program_id(0)
        slot = jax.lax.rem(i, 2)

        @pl.when(i == 0)
        def _():
            _copy(emb_ref, buf, sems, 0, 0, n_blocks == 1).start()

        @pl.when(i + 1 < n_blocks)
        def _():
            nxt = 1 - slot

            @pl.when(i + 2 < n_blocks)
            def _():
                _copy(emb_ref, buf, sems, i + 1, nxt, False).start()

            @pl.when(i + 2 >= n_blocks)
            def _():
                _copy(emb_ref, buf, sems, i + 1, nxt, True).start()

        @pl.when(i + 1 < n_blocks)
        def _():
            _copy(emb_ref, buf, sems, i, slot, False).wait()

        @pl.when(i + 1 >= n_blocks)
        def _():
            _copy(emb_ref, buf, sems, i, slot, n_blocks >= 1).wait()

        su_b = su_ref[...].astype(jnp.bfloat16)
        cu_b = cu_ref[...].astype(jnp.bfloat16)
        wq = wq_ref[...]
        w = wq[:, :2 * _D]
        sa2 = jnp.sum(wq[:, 2 * _D:], axis=1, keepdims=True)
        pw = pw_ref[0, 0]
        mw = mw_ref[0, 0]
        tq = _TN // _Q
        for q in range(_Q):
            tail = buf[slot, pl.ds(q * tq, tq), :]
            sig = _score_quarter(tail, su_b, cu_b, w, sa2, pw, mw)
            out_ref[:, q * tq:(q + 1) * tq] = sig

    return _score_body


@functools.partial(jax.jit, static_argnums=(0,))
def _run(num_ents, e1, rel, emb_e, emb_rel, phase_weight, modulus_weight):
    emb_e3 = emb_e.reshape(num_ents, 1, 2 * _D)
    emb_rel3 = emb_rel.reshape(emb_rel.shape[0], 1, 3 * _D)
    su, cu, wq = pl.pallas_call(
        _precompute_body,
        grid_spec=pltpu.PrefetchScalarGridSpec(
            num_scalar_prefetch=2,
            grid=(_B,),
            in_specs=[
                pl.BlockSpec((1, 1, 2 * _D), lambda b, e1r, relr: (e1r[b], 0, 0)),
                pl.BlockSpec((1, 1, 3 * _D), lambda b, e1r, relr: (relr[b], 0, 0)),
            ],
            out_specs=[
                pl.BlockSpec((1, 1, _D), lambda b, e1r, relr: (b, 0, 0)),
                pl.BlockSpec((1, 1, _D), lambda b, e1r, relr: (b, 0, 0)),
                pl.BlockSpec((1, 1, 3 * _D), lambda b, e1r, relr: (b, 0, 0)),
            ],
        ),
        out_shape=[
            jax.ShapeDtypeStruct((_B, 1, _D), jnp.float32),
            jax.ShapeDtypeStruct((_B, 1, _D), jnp.float32),
            jax.ShapeDtypeStruct((_B, 1, 3 * _D), jnp.float32),
        ],
    )(e1, rel, emb_e3, emb_rel3)
    su = su.reshape(_B, _D)
    cu = cu.reshape(_B, _D)
    wq = wq.reshape(_B, 3 * _D)

    grid = (num_ents + _TN - 1) // _TN
    out = pl.pallas_call(
        _make_score_body(num_ents),
        grid=(grid,),
        in_specs=[
            pl.BlockSpec((_B, _D), lambda i: (0, 0)),
            pl.BlockSpec((_B, _D), lambda i: (0, 0)),
            pl.BlockSpec((_B, 3 * _D), lambda i: (0, 0)),
            pl.BlockSpec(memory_space=pltpu.SMEM),
            pl.BlockSpec(memory_space=pltpu.SMEM),
            pl.BlockSpec(memory_space=pltpu.MemorySpace.HBM),
        ],
        out_specs=pl.BlockSpec((_B, _TN), lambda i: (0, i)),
        out_shape=jax.ShapeDtypeStruct((_B, num_ents), jnp.float32),
        scratch_shapes=[
            pltpu.VMEM((2, _TN, 2 * _D), jnp.float32),
            pltpu.SemaphoreType.DMA((2,)),
        ],
    )(su, cu, wq, phase_weight, modulus_weight, emb_e)
    return out


def kernel(g, e1, rel, e2_multi, emb_e, emb_rel, phase_weight, modulus_weight):
    return _run(emb_e.shape[0], e1, rel, emb_e, emb_rel,
                phase_weight, modulus_weight)
